# Initial kernel scaffold; baseline (speedup 1.0000x reference)
import jax
import jax.numpy as jnp
from jax.experimental import pallas as pl


def kernel(static, runoff, Q_prev, I_prev, edges, node_mask, edge_mask, W1, b1, W2, b2, W3, b3):
    raise NotImplementedError("write your pallas kernel here")


# speedup vs baseline: 21.9846x; 21.9846x over previous
import functools

import jax
import jax.numpy as jnp
from jax import lax
from jax.experimental import pallas as pl
from jax.experimental.pallas import tpu as pltpu
from jax.experimental.pallas import tpu_sc as plsc

_N = 10000
_E = 320000
_NPAD = 10240
_NW = 32
_EPW = _E // _NW
_SLICE = _NPAD // 16
_BM = 1024


def _sc_scatter_add(src, dst, emask, q_pad):
  mesh = plsc.VectorSubcoreMesh(core_axis_name="c", subcore_axis_name="s")

  @functools.partial(
      pl.kernel,
      out_type=jax.ShapeDtypeStruct((2, _NPAD), jnp.float32),
      mesh=mesh,
      compiler_params=pltpu.CompilerParams(needs_layout_passes=False),
      scratch_types=[
          pltpu.VMEM((_EPW,), jnp.int32),
          pltpu.VMEM((_EPW,), jnp.int32),
          pltpu.VMEM((_EPW,), jnp.float32),
          pltpu.VMEM((_NPAD,), jnp.float32),
          pltpu.VMEM((_NPAD,), jnp.float32),
          pltpu.VMEM_SHARED((16, _NPAD), jnp.float32),
          pltpu.VMEM((16, _SLICE), jnp.float32),
      ],
  )
  def k(src_hbm, dst_hbm, m_hbm, q_hbm, out_hbm,
        src_v, dst_v, m_v, q_v, acc_v, shared, red_v):
    cid = lax.axis_index("c")
    sid = lax.axis_index("s")
    wid = cid * 16 + sid
    base = wid * _EPW
    pltpu.sync_copy(src_hbm.at[pl.ds(base, _EPW)], src_v)
    pltpu.sync_copy(dst_hbm.at[pl.ds(base, _EPW)], dst_v)
    pltpu.sync_copy(m_hbm.at[pl.ds(base, _EPW)], m_v)
    pltpu.sync_copy(q_hbm, q_v)

    zero = jnp.zeros((16,), jnp.float32)

    def zbody(i, carry):
      acc_v[pl.ds(i * 16, 16)] = zero
      return carry

    lax.fori_loop(0, _NPAD // 16, zbody, 0)

    def ebody(i, carry):
      o = i * 16
      s_idx = src_v[pl.ds(o, 16)]
      d_idx = dst_v[pl.ds(o, 16)]
      vals = plsc.load_gather(q_v, [s_idx]) * m_v[pl.ds(o, 16)]
      plsc.addupdate_scatter(acc_v, [d_idx], vals)
      return carry

    lax.fori_loop(0, _EPW // 16, ebody, 0)

    pltpu.sync_copy(acc_v, shared.at[sid])
    plsc.subcore_barrier()
    col = sid * _SLICE
    pltpu.sync_copy(shared.at[:, pl.ds(col, _SLICE)], red_v)

    def rbody(i, carry):
      o = i * 16
      a = red_v[0, pl.ds(o, 16)]
      for r in range(1, 16):
        a = a + red_v[r, pl.ds(o, 16)]
      acc_v[pl.ds(o, 16)] = a
      return carry

    lax.fori_loop(0, _SLICE // 16, rbody, 0)
    pltpu.sync_copy(acc_v.at[pl.ds(0, _SLICE)], out_hbm.at[cid, pl.ds(col, _SLICE)])

  return k(src, dst, emask, q_pad)


def _tc_body(st_ref, qp_ref, r_ref, ip_ref, nm_ref, up_ref,
             w1_ref, wq_ref, wr_ref, b1_ref, w2_ref, b2_ref,
             w30_ref, w31_ref, b30_ref, b31_ref,
             qout_ref, iout_ref):
  x = st_ref[...]
  qp8 = qp_ref[...]
  r8 = r_ref[...]

  h = jnp.dot(x, w1_ref[...], preferred_element_type=jnp.float32)
  h = h + qp8 * wq_ref[...] + r8 * wr_ref[...] + b1_ref[...]
  h = jnp.maximum(h, 0.0)
  h = jnp.dot(h, w2_ref[...], preferred_element_type=jnp.float32) + b2_ref[...]
  h = jnp.maximum(h, 0.0)

  p0 = jnp.sum(h * w30_ref[...], axis=1, keepdims=True) + b30_ref[...]
  p1 = jnp.sum(h * w31_ref[...], axis=1, keepdims=True) + b31_ref[...]

  k2 = 2.0 * (jnp.maximum(p0, 0.0) + jnp.log1p(jnp.exp(-jnp.abs(p0))))
  t = k2 / (1.0 + jnp.exp(-p1)) * 0.5
  u = k2 - t
  inv = 1.0 / (u + 1.0)

  up = up_ref[0] + up_ref[1]
  i_curr = up + r8
  q_curr = ((1.0 - t) * i_curr + (1.0 + t) * ip_ref[...]
            + (u - 1.0) * qp8 + 2.0 * r8) * inv
  nm = nm_ref[...]
  qout_ref[...] = q_curr * nm
  iout_ref[...] = i_curr * nm


def _tc_mlp(static_pad, qp2, r2, ip2, nm2, up2,
            w1st, wq, wr, b1, w2t, b2, w30, w31, b30, b31):
  grid = (_NPAD // _BM,)
  col = pl.BlockSpec((_BM, 1), lambda i: (i, 0))
  full = lambda shape: pl.BlockSpec(shape, lambda i: tuple(0 for _ in shape))
  return pl.pallas_call(
      _tc_body,
      grid=grid,
      in_specs=[
          pl.BlockSpec((_BM, 128), lambda i: (i, 0)),
          col, col, col, col,
          pl.BlockSpec((2, _BM, 1), lambda i: (0, i, 0)),
          full((128, 256)), full((1, 256)), full((1, 256)), full((1, 256)),
          full((256, 256)), full((1, 256)),
          full((1, 256)), full((1, 256)), full((1, 1)), full((1, 1)),
      ],
      out_specs=[col, col],
      out_shape=[
          jax.ShapeDtypeStruct((_NPAD, 1), jnp.float32),
          jax.ShapeDtypeStruct((_NPAD, 1), jnp.float32),
      ],
  )(static_pad, qp2, r2, ip2, nm2, up2,
    w1st, wq, wr, b1, w2t, b2, w30, w31, b30, b31)


def kernel(static, runoff, Q_prev, I_prev, edges, node_mask, edge_mask,
           W1, b1, W2, b2, W3, b3):
  n = runoff.shape[0]
  pad = _NPAD - n
  src = edges[0]
  dst = edges[1]

  q_pad = jnp.pad(Q_prev, (0, pad))
  r_pad = jnp.pad(runoff, (0, pad))
  i_pad = jnp.pad(I_prev, (0, pad))
  nm_pad = jnp.pad(node_mask.astype(jnp.float32), (0, pad))
  static_pad = jnp.pad(static, ((0, pad), (0, 0)))

  partials = _sc_scatter_add(src, dst, edge_mask, q_pad)

  qp2 = q_pad.reshape(_NPAD, 1)
  r2 = r_pad.reshape(_NPAD, 1)
  ip2 = i_pad.reshape(_NPAD, 1)
  nm2 = nm_pad.reshape(_NPAD, 1)
  up2 = partials.reshape(2, _NPAD, 1)

  w1st = W1[:, :128].T
  wq = W1[:, 128].reshape(1, 256)
  wr = W1[:, 129].reshape(1, 256)
  w2t = W2.T
  w30 = W3[0].reshape(1, 256)
  w31 = W3[1].reshape(1, 256)
  b30 = b3[0].reshape(1, 1)
  b31 = b3[1].reshape(1, 1)

  q_out, i_out = _tc_mlp(static_pad, qp2, r2, ip2, nm2, up2,
                         w1st, wq, wr, b1.reshape(1, 256), w2t,
                         b2.reshape(1, 256), w30, w31, b30, b31)
  return (q_out.reshape(_NPAD)[:n], i_out.reshape(_NPAD)[:n])
